# Initial kernel scaffold; baseline (speedup 1.0000x reference)
import jax
import jax.numpy as jnp
from jax.experimental import pallas as pl


def kernel(x, keys, labels):
    raise NotImplementedError("write your pallas kernel here")


# speedup vs baseline: 3.4191x; 3.4191x over previous
import dataclasses
import functools

import jax
import jax.numpy as jnp
from jax import lax
from jax.experimental import pallas as pl
from jax.experimental.pallas import tpu as pltpu
from jax.experimental.pallas import tpu_sc as plsc

Q = 1024
D = 64
K = 100000
KB = 2500
NB = K // KB
IBIG = 2**30


def _nn_body(x_ref, xsq_ref, keys_ref, ksq_ref, out_ref, rmin_ref, ridx_ref):
    i = pl.program_id(0)
    kb = keys_ref[0]
    ksq = ksq_ref[0]
    m = lax.dot_general(
        x_ref[...], kb,
        dimension_numbers=(((1,), (1,)), ((), ())),
        preferred_element_type=jnp.float32,
    )
    d = (xsq_ref[...] + ksq) - 2.0 * m
    bmin = jnp.min(d, axis=1, keepdims=True)
    iota = lax.broadcasted_iota(jnp.int32, d.shape, 1)
    bidx = jnp.min(jnp.where(d == bmin, iota, IBIG), axis=1,
                   keepdims=True) + i * KB

    @pl.when(i == 0)
    def _():
        rmin_ref[...] = bmin
        ridx_ref[...] = bidx

    @pl.when(i != 0)
    def _():
        prev = rmin_ref[...]
        upd = bmin < prev
        rmin_ref[...] = jnp.where(upd, bmin, prev)
        ridx_ref[...] = jnp.where(upd, bidx, ridx_ref[...])

    @pl.when(i == NB - 1)
    def _():
        out_ref[...] = ridx_ref[...]


def _nn_argmin(x, xsq, keys3, ksq3, interpret=False):
    return pl.pallas_call(
        _nn_body,
        grid=(NB,),
        in_specs=[
            pl.BlockSpec((Q, D), lambda i: (0, 0)),
            pl.BlockSpec((Q, 1), lambda i: (0, 0)),
            pl.BlockSpec((1, KB, D), lambda i: (i, 0, 0)),
            pl.BlockSpec((1, 1, KB), lambda i: (i, 0, 0)),
        ],
        out_specs=pl.BlockSpec((Q, 1), lambda i: (0, 0)),
        out_shape=jax.ShapeDtypeStruct((Q, 1), jnp.int32),
        scratch_shapes=[
            pltpu.VMEM((Q, 1), jnp.float32),
            pltpu.VMEM((Q, 1), jnp.int32),
        ],
        interpret=interpret,
    )(x, xsq, keys3, ksq3)


def _sc_compiler_params():
    cp = pltpu.CompilerParams()
    if "needs_layout_passes" in pltpu.CompilerParams.__dataclass_fields__:
        cp = dataclasses.replace(cp, needs_layout_passes=False)
    return cp


def _label_gather(labels, nn_idx):
    mesh = plsc.VectorSubcoreMesh(core_axis_name="c", subcore_axis_name="s")

    @functools.partial(
        pl.kernel,
        mesh=mesh,
        out_type=jax.ShapeDtypeStruct((Q,), labels.dtype),
        scratch_types=[
            pltpu.VMEM((K,), labels.dtype),
            pltpu.VMEM((Q,), jnp.int32),
            pltpu.VMEM((Q,), labels.dtype),
        ],
        compiler_params=_sc_compiler_params(),
    )
    def gather_kernel(labels_hbm, idx_hbm, out_hbm, lab_v, idx_v, out_v):
        cid = lax.axis_index("c")
        sid = lax.axis_index("s")

        @pl.when(jnp.logical_and(cid == 0, sid == 0))
        def _():
            pltpu.sync_copy(labels_hbm, lab_v)
            pltpu.sync_copy(idx_hbm, idx_v)
            for j in range(Q // 16):
                ids = idx_v[pl.ds(j * 16, 16)]
                out_v[pl.ds(j * 16, 16)] = plsc.load_gather(lab_v, [ids])
            pltpu.sync_copy(out_v, out_hbm)

    return gather_kernel(labels, nn_idx)


def kernel(x, keys, labels):
    xsq = jnp.sum(x * x, axis=1, keepdims=True)
    ksq = jnp.sum(keys * keys, axis=1)
    keys3 = keys.reshape(NB, KB, D)
    ksq3 = ksq.reshape(NB, 1, KB)
    nn_idx = _nn_argmin(x, xsq, keys3, ksq3)
    return _label_gather(labels, nn_idx.reshape(Q))
